# baseline (speedup 1.0000x reference)
import functools

import jax
import jax.numpy as jnp
from jax import lax
from jax.experimental import pallas as pl
from jax.experimental.pallas import tpu as pltpu
from jax.experimental.pallas import tpu_sc as plsc

_NUM_USERS = 1000000
_NUM_FIELDS = 26
_CAT_VOCAB = 100000
_EMB = 64
_DM = 512
_B = 4096
_F_ALL = _NUM_FIELDS + 1
_IN_DIM = _F_ALL * _EMB
_XW = 1792

_NC, _NS = 2, 16
_NW = _NC * _NS
_BPW = _B // _NW
_SUB = 32


def _gather_body(gr_hbm, uid_hbm, feat_hbm, x_hbm,
                 gr_v, rows_v, stage_v, sem):
    wid = lax.axis_index("s") * _NC + lax.axis_index("c")
    base = wid * _BPW

    def issue(tab_hbm, f, pb):
        for blk in range(_SUB // 16):
            idx16 = (lax.iota(jnp.int32, 16) + blk * 16) * _F_ALL + f
            v16 = plsc.load_gather(gr_v, [idx16])
            for j in range(16):
                r = v16[j]
                col = pl.multiple_of(((r >> 1) & 3) * 128, 128)
                pltpu.async_copy(
                    tab_hbm.at[r >> 3, pl.ds(col, 128)],
                    rows_v.at[pb, blk * 16 + j], sem)

    def drain_copyout(fprev, pb):
        pltpu.make_async_copy(
            feat_hbm.at[pl.ds(0, _SUB), pl.ds(0, 128)], rows_v.at[pb], sem
        ).wait()
        for blk in range(_SUB // 16):
            idx16 = (lax.iota(jnp.int32, 16) + blk * 16) * _F_ALL + fprev
            v16 = plsc.load_gather(gr_v, [idx16])
            for j in range(16):
                i = blk * 16 + j
                off = pl.multiple_of((v16[j] & 1) * _EMB, _EMB)
                for k in range(_EMB // 16):
                    stage_v[i, pl.ds(fprev * _EMB + k * 16, 16)] = (
                        rows_v[pb, i, pl.ds(off + k * 16, 16)])

    @pl.loop(0, _BPW // _SUB)
    def _sub(s):
        b0 = base + s * _SUB

        pltpu.sync_copy(gr_hbm.at[pl.ds(b0 * _F_ALL, _SUB * _F_ALL)], gr_v)

        @pl.loop(0, _SUB)
        def _z(i):
            for k in range(_EMB // 16):
                stage_v[i, pl.ds(_IN_DIM + k * 16, 16)] = (
                    jnp.zeros((16,), jnp.float32))

        issue(uid_hbm, 0, 0)

        @pl.loop(1, _F_ALL + 1)
        def _field(f):
            @pl.when(f < _F_ALL)
            def _():
                issue(feat_hbm, f, f & 1)
            drain_copyout(f - 1, (f - 1) & 1)

        pltpu.sync_copy(stage_v, x_hbm.at[pl.ds(b0, _SUB), :])


@functools.cache
def _make_gather():
    return pl.kernel(
        _gather_body,
        out_type=jax.ShapeDtypeStruct((_B, _XW), jnp.float32),
        mesh=plsc.VectorSubcoreMesh(core_axis_name="c", subcore_axis_name="s",
                                    num_cores=_NC, num_subcores=_NS),
        scratch_types=[
            pltpu.VMEM((_SUB * _F_ALL,), jnp.int32),
            pltpu.VMEM((2, _SUB, 2 * _EMB), jnp.float32),
            pltpu.VMEM((_SUB, _XW), jnp.float32),
            pltpu.SemaphoreType.DMA,
        ],
        compiler_params=pltpu.CompilerParams(needs_layout_passes=False),
    )


_BM = 256


def _mlp_body(x_ref, w1_ref, b1_ref, w2_ref, b2_ref, o_ref):
    h = jnp.dot(x_ref[...], w1_ref[...], preferred_element_type=jnp.float32)
    h = h + b1_ref[...]
    h = jnp.where(h >= 0, h, 0.01 * h)
    o = jnp.dot(h, w2_ref[...], preferred_element_type=jnp.float32)
    o_ref[...] = o + b2_ref[...]


def kernel(uid, onehot_feats, uid_table, feat_tables, W1, b1, W2, b2):
    gr = jnp.concatenate(
        [uid[:, None],
         onehot_feats + (jnp.arange(_NUM_FIELDS, dtype=jnp.int32)
                         * _CAT_VOCAB)[None, :]],
        axis=1).reshape(_F_ALL * _B)
    uid_p = uid_table.reshape(_NUM_USERS // 8, 8 * _EMB)
    feat_p = feat_tables.reshape(_NUM_FIELDS * _CAT_VOCAB // 8, 8 * _EMB)

    x = _make_gather()(gr, uid_p, feat_p)

    W1p = jnp.pad(W1, ((0, _XW - _IN_DIM), (0, 0)))
    out = pl.pallas_call(
        _mlp_body,
        grid=(_B // _BM,),
        in_specs=[
            pl.BlockSpec((_BM, _XW), lambda i: (i, 0)),
            pl.BlockSpec((_XW, _DM), lambda i: (0, 0)),
            pl.BlockSpec((1, _DM), lambda i: (0, 0)),
            pl.BlockSpec((_DM, _DM), lambda i: (0, 0)),
            pl.BlockSpec((1, _DM), lambda i: (0, 0)),
        ],
        out_specs=pl.BlockSpec((_BM, _DM), lambda i: (i, 0)),
        out_shape=jax.ShapeDtypeStruct((_B, _DM), jnp.float32),
    )(x, W1p, b1.reshape(1, _DM), W2, b2.reshape(1, _DM))
    return out[:, None, :]


# speedup vs baseline: 1.5150x; 1.5150x over previous
import functools

import jax
import jax.numpy as jnp
from jax import lax
from jax.experimental import pallas as pl
from jax.experimental.pallas import tpu as pltpu
from jax.experimental.pallas import tpu_sc as plsc

_NUM_USERS = 1000000
_NUM_FIELDS = 26
_CAT_VOCAB = 100000
_EMB = 64
_DM = 512
_B = 4096
_F_ALL = _NUM_FIELDS + 1
_IN_DIM = _F_ALL * _EMB
_XW = 1792

_NC, _NS = 2, 16
_NW = _NC * _NS
_BPW = _B // _NW
_SUB = 32
_FA = 6


def _gather_body(gr_hbm, uid_hbm, feata_hbm, featb_hbm, x_hbm,
                 gr_v, rows_v, stage_v, sem):
    wid = lax.axis_index("s") * _NC + lax.axis_index("c")
    base = wid * _BPW

    def issue(row_at, f, pb):
        for blk in range(_SUB // 16):
            idx16 = (lax.iota(jnp.int32, 16) + blk * 16) * _F_ALL + f
            v16 = plsc.load_gather(gr_v, [idx16])
            for j in range(16):
                pltpu.async_copy(
                    row_at(v16[j]), rows_v.at[pb, blk * 16 + j], sem)

    def drain_copyout(fprev, pb):
        pltpu.make_async_copy(
            featb_hbm.at[pl.ds(0, _SUB)], rows_v.at[pb], sem
        ).wait()
        for i in range(_SUB):
            for k in range(_EMB // 16):
                stage_v[i, pl.ds(fprev * _EMB + k * 16, 16)] = (
                    rows_v[pb, i, pl.ds(k * 16, 16)])

    @pl.loop(0, _BPW // _SUB)
    def _sub(s):
        b0 = base + s * _SUB

        pltpu.sync_copy(gr_hbm.at[pl.ds(b0 * _F_ALL, _SUB * _F_ALL)], gr_v)

        @pl.loop(0, _SUB)
        def _z(i):
            for k in range(_EMB // 16):
                stage_v[i, pl.ds(_IN_DIM + k * 16, 16)] = (
                    jnp.zeros((16,), jnp.float32))

        issue(lambda r: uid_hbm.at[r], 0, 0)

        @pl.loop(1, _F_ALL + 1)
        def _field(f):
            @pl.when(f <= _FA)
            def _():
                issue(lambda r: feata_hbm.at[f - 1, r], f, f & 1)

            @pl.when((f > _FA) & (f < _F_ALL))
            def _():
                issue(lambda r: featb_hbm.at[r], f, f & 1)

            drain_copyout(f - 1, (f - 1) & 1)

        pltpu.sync_copy(stage_v, x_hbm.at[pl.ds(b0, _SUB), :])


@functools.cache
def _make_gather():
    return pl.kernel(
        _gather_body,
        out_type=jax.ShapeDtypeStruct((_B, _XW), jnp.float32),
        mesh=plsc.VectorSubcoreMesh(core_axis_name="c", subcore_axis_name="s",
                                    num_cores=_NC, num_subcores=_NS),
        scratch_types=[
            pltpu.VMEM((_SUB * _F_ALL,), jnp.int32),
            pltpu.VMEM((2, _SUB, _EMB), jnp.float32),
            pltpu.VMEM((_SUB, _XW), jnp.float32),
            pltpu.SemaphoreType.DMA,
        ],
        compiler_params=pltpu.CompilerParams(needs_layout_passes=False),
    )


_BM = 256


def _mlp_body(x_ref, w1_ref, b1_ref, w2_ref, b2_ref, o_ref):
    h = jnp.dot(x_ref[...], w1_ref[...], preferred_element_type=jnp.float32)
    h = h + b1_ref[...]
    h = jnp.where(h >= 0, h, 0.01 * h)
    o = jnp.dot(h, w2_ref[...], preferred_element_type=jnp.float32)
    o_ref[...] = o + b2_ref[...]


def kernel(uid, onehot_feats, uid_table, feat_tables, W1, b1, W2, b2):
    offs = jnp.where(jnp.arange(_NUM_FIELDS, dtype=jnp.int32) < _FA, 0,
                     (jnp.arange(_NUM_FIELDS, dtype=jnp.int32) - _FA)
                     * _CAT_VOCAB)
    gr = jnp.concatenate(
        [uid[:, None], onehot_feats + offs[None, :]],
        axis=1).reshape(_F_ALL * _B)
    feat_a = feat_tables[:_FA]
    feat_b = feat_tables[_FA:].reshape((_NUM_FIELDS - _FA) * _CAT_VOCAB,
                                       _EMB)

    x = _make_gather()(gr, uid_table, feat_a, feat_b)

    W1p = jnp.pad(W1, ((0, _XW - _IN_DIM), (0, 0)))
    out = pl.pallas_call(
        _mlp_body,
        grid=(_B // _BM,),
        in_specs=[
            pl.BlockSpec((_BM, _XW), lambda i: (i, 0)),
            pl.BlockSpec((_XW, _DM), lambda i: (0, 0)),
            pl.BlockSpec((1, _DM), lambda i: (0, 0)),
            pl.BlockSpec((_DM, _DM), lambda i: (0, 0)),
            pl.BlockSpec((1, _DM), lambda i: (0, 0)),
        ],
        out_specs=pl.BlockSpec((_BM, _DM), lambda i: (i, 0)),
        out_shape=jax.ShapeDtypeStruct((_B, _DM), jnp.float32),
    )(x, W1p, b1.reshape(1, _DM), W2, b2.reshape(1, _DM))
    return out[:, None, :]


# speedup vs baseline: 2.4139x; 1.5934x over previous
import functools

import jax
import jax.numpy as jnp
from jax import lax
from jax.experimental import pallas as pl
from jax.experimental.pallas import tpu as pltpu
from jax.experimental.pallas import tpu_sc as plsc

_NUM_USERS = 1000000
_NUM_FIELDS = 26
_CAT_VOCAB = 100000
_EMB = 64
_DM = 512
_B = 4096
_F_ALL = _NUM_FIELDS + 1
_IN_DIM = _F_ALL * _EMB
_XW = 1792

_NC, _NS = 2, 16
_NW = _NC * _NS
_BPW = _B // _NW
_SUB = 32


def _gather_body(gr_hbm, uid_hbm, feat_hbm, x_hbm,
                 gr_v, rows_v, stage_v, sem):
    wid = lax.axis_index("s") * _NC + lax.axis_index("c")
    base = wid * _BPW

    def issue(tab_hbm, f, pb):
        for blk in range(_SUB // 16):
            idx16 = (lax.iota(jnp.int32, 16) + blk * 16) * _F_ALL + f
            v16 = plsc.load_gather(gr_v, [idx16])
            for j in range(16):
                pltpu.async_copy(
                    tab_hbm.at[v16[j]], rows_v.at[pb, blk * 16 + j], sem)

    def drain_copyout(fprev, pb):
        pltpu.make_async_copy(
            feat_hbm.at[pl.ds(0, _SUB)], rows_v.at[pb], sem
        ).wait()
        for i in range(_SUB):
            for k in range(_EMB // 16):
                stage_v[i, pl.ds(fprev * _EMB + k * 16, 16)] = (
                    rows_v[pb, i, pl.ds(k * 16, 16)])

    @pl.loop(0, _BPW // _SUB)
    def _sub(s):
        b0 = base + s * _SUB

        pltpu.sync_copy(gr_hbm.at[pl.ds(b0 * _F_ALL, _SUB * _F_ALL)], gr_v)

        @pl.loop(0, _SUB)
        def _z(i):
            for k in range(_EMB // 16):
                stage_v[i, pl.ds(_IN_DIM + k * 16, 16)] = (
                    jnp.zeros((16,), jnp.float32))

        issue(uid_hbm, 0, 0)

        @pl.loop(1, _F_ALL + 1)
        def _field(f):
            @pl.when(f < _F_ALL)
            def _():
                issue(feat_hbm, f, f & 1)
            drain_copyout(f - 1, (f - 1) & 1)

        pltpu.sync_copy(stage_v, x_hbm.at[pl.ds(b0, _SUB), :])


@functools.cache
def _make_gather():
    return pl.kernel(
        _gather_body,
        out_type=jax.ShapeDtypeStruct((_B, _XW), jnp.float32),
        mesh=plsc.VectorSubcoreMesh(core_axis_name="c", subcore_axis_name="s",
                                    num_cores=_NC, num_subcores=_NS),
        scratch_types=[
            pltpu.VMEM((_SUB * _F_ALL,), jnp.int32),
            pltpu.VMEM((2, _SUB, _EMB), jnp.float32),
            pltpu.VMEM((_SUB, _XW), jnp.float32),
            pltpu.SemaphoreType.DMA,
        ],
        compiler_params=pltpu.CompilerParams(needs_layout_passes=False),
    )


_BM = 256


def _mlp_body(x_ref, w1_ref, b1_ref, w2_ref, b2_ref, o_ref):
    h = jnp.dot(x_ref[...], w1_ref[...], preferred_element_type=jnp.float32)
    h = h + b1_ref[...]
    h = jnp.where(h >= 0, h, 0.01 * h)
    o = jnp.dot(h, w2_ref[...], preferred_element_type=jnp.float32)
    o_ref[...] = o + b2_ref[...]


def kernel(uid, onehot_feats, uid_table, feat_tables, W1, b1, W2, b2):
    gr = jnp.concatenate(
        [uid[:, None],
         onehot_feats + (jnp.arange(_NUM_FIELDS, dtype=jnp.int32)
                         * _CAT_VOCAB)[None, :]],
        axis=1).reshape(_F_ALL * _B)
    feat_flat = feat_tables.reshape(_NUM_FIELDS * _CAT_VOCAB, _EMB)

    x = _make_gather()(gr, uid_table, feat_flat)

    W1p = jnp.pad(W1, ((0, _XW - _IN_DIM), (0, 0)))
    out = pl.pallas_call(
        _mlp_body,
        grid=(_B // _BM,),
        in_specs=[
            pl.BlockSpec((_BM, _XW), lambda i: (i, 0)),
            pl.BlockSpec((_XW, _DM), lambda i: (0, 0)),
            pl.BlockSpec((1, _DM), lambda i: (0, 0)),
            pl.BlockSpec((_DM, _DM), lambda i: (0, 0)),
            pl.BlockSpec((1, _DM), lambda i: (0, 0)),
        ],
        out_specs=pl.BlockSpec((_BM, _DM), lambda i: (i, 0)),
        out_shape=jax.ShapeDtypeStruct((_B, _DM), jnp.float32),
    )(x, W1p, b1.reshape(1, _DM), W2, b2.reshape(1, _DM))
    return out[:, None, :]
